# baseline (speedup 1.0000x reference)
import functools

import jax
import jax.numpy as jnp
from jax import lax
from jax.experimental import pallas as pl
from jax.experimental.pallas import tpu as pltpu
from jax.experimental.pallas import tpu_sc as plsc

H = 128
G = 16
ROW_BLK = 512
K = 128
NC = 2
NS = 16
NW = NC * NS


def _cdiv(a, b):
    return (a + b - 1) // b


def _make_gather(n_tbl, npad):
    rows_w = npad // NW
    ck = next(d for d in range(min(K, rows_w), 0, -8)
              if rows_w % d == 0)
    n_ch = rows_w // ck

    def body(tbl, ids, out, idx_v, rows_v, sem):
        c = lax.axis_index("c")
        s = lax.axis_index("s")
        base = (c * NS + s) * rows_w
        for i in range(n_ch):
            off = base + i * ck
            pltpu.sync_copy(ids.at[pl.ds(off, ck)], idx_v)
            pltpu.async_copy(tbl.at[idx_v], rows_v, sem).wait()
            pltpu.sync_copy(rows_v, out.at[pl.ds(off, ck)])

    return pl.kernel(
        body,
        mesh=plsc.VectorSubcoreMesh(core_axis_name="c", subcore_axis_name="s"),
        out_type=jax.ShapeDtypeStruct((npad, H), jnp.float32),
        scratch_types=[
            pltpu.VMEM((ck,), jnp.int32),
            pltpu.VMEM((ck, H), jnp.float32),
            pltpu.SemaphoreType.DMA,
        ],
    )


def _make_scatter(n, npad, epad):
    KS = 64
    per_w = epad // NW
    n_ch = per_w // KS
    nr = _cdiv(n + 1, NS * 8) * NS * 8
    rows_t = nr // NS
    zchunks = [(i * KS, KS) for i in range(rows_t // KS)]
    if rows_t % KS:
        zchunks.append((rows_t - rows_t % KS, rows_t % KS))

    assert n_ch % 2 == 0
    n_pair = n_ch // 2

    def body(hm_tbl, rm_tbl, srcr, dstr, relr, agg_out,
             idx_s0, idx_d0, idx_r0, buf_a0, buf_b0, sem_a0, sem_b0,
             idx_s1, idx_d1, idx_r1, buf_a1, buf_b1, sem_a1, sem_b1,
             agg):
        c = lax.axis_index("c")
        s = lax.axis_index("s")
        slots = [(idx_s0, idx_d0, idx_r0, buf_a0, buf_b0, sem_a0, sem_b0),
                 (idx_s1, idx_d1, idx_r1, buf_a1, buf_b1, sem_a1, sem_b1)]
        zeros16 = jnp.zeros((16,), jnp.float32)

        def zrow(i, carry):
            for j in range(H // 16):
                buf_a0[i, pl.ds(j * 16, 16)] = zeros16
            return carry
        lax.fori_loop(0, KS, zrow, 0)

        for zoff, zlen in zchunks:
            r0 = s * rows_t + zoff
            pltpu.sync_copy(buf_a0.at[pl.ds(0, zlen)], agg.at[pl.ds(r0, zlen)])
        plsc.subcore_barrier()

        base = (c * NS + s) * per_w

        def load(sl, off):
            idx_s, idx_d, idx_r, buf_a, buf_b, sem_a, sem_b = sl
            pltpu.sync_copy(srcr.at[pl.ds(off, KS)], idx_s)
            pltpu.sync_copy(relr.at[pl.ds(off, KS)], idx_r)
            pltpu.sync_copy(dstr.at[pl.ds(off, KS)], idx_d)
            pltpu.async_copy(hm_tbl.at[idx_s], buf_a, sem_a)
            pltpu.async_copy(rm_tbl.at[idx_r], buf_b, sem_b)

        def wait(sl):
            idx_s, idx_d, idx_r, buf_a, buf_b, sem_a, sem_b = sl
            pltpu.make_async_copy(hm_tbl.at[idx_s], buf_a, sem_a).wait()
            pltpu.make_async_copy(rm_tbl.at[idx_r], buf_b, sem_b).wait()

        def scat(sl):
            idx_s, idx_d, idx_r, buf_a, buf_b, sem_a, sem_b = sl

            def addrow(i, carry):
                for j in range(H // 16):
                    sl_j = pl.ds(j * 16, 16)
                    buf_a[i, sl_j] = buf_a[i, sl_j] + buf_b[i, sl_j]
                return carry
            lax.fori_loop(0, KS, addrow, 0)
            pltpu.sync_copy(buf_a, agg.at[idx_d], add=True)

        load(slots[0], base)

        def pair(j, carry):
            off = base + j * 2 * KS
            load(slots[1], off + KS)
            wait(slots[0])
            scat(slots[0])
            load(slots[0], off + 2 * KS)
            wait(slots[1])
            scat(slots[1])
            return carry
        lax.fori_loop(0, n_pair, pair, 0)
        wait(slots[0])
        plsc.subcore_barrier()

        r0 = s * rows_t
        pltpu.sync_copy(agg.at[pl.ds(r0, rows_t)],
                        agg_out.at[c, pl.ds(r0, rows_t)])

    slot_types = [
        pltpu.VMEM((KS,), jnp.int32),
        pltpu.VMEM((KS,), jnp.int32),
        pltpu.VMEM((KS,), jnp.int32),
        pltpu.VMEM((KS, H), jnp.float32),
        pltpu.VMEM((KS, H), jnp.float32),
        pltpu.SemaphoreType.DMA,
        pltpu.SemaphoreType.DMA,
    ]
    return pl.kernel(
        body,
        mesh=plsc.VectorSubcoreMesh(core_axis_name="c", subcore_axis_name="s"),
        out_type=jax.ShapeDtypeStruct((NC, npad, H), jnp.float32),
        scratch_types=slot_types + slot_types + [
            pltpu.VMEM_SHARED((nr, H), jnp.float32),
        ],
    )


def _make_degree(n, npad, epad):
    per_w = epad // NW
    n_ch = per_w // K
    nr = _cdiv(n + 1, NS * 8) * NS * 8
    rows_t = nr // NS
    zchunks = [(i * K, K) for i in range(rows_t // K)]
    if rows_t % K:
        zchunks.append((rows_t - rows_t % K, rows_t % K))

    def body(dstr, deg_out, idx_d, ones_v, deg):
        c = lax.axis_index("c")
        s = lax.axis_index("s")
        zeros16 = jnp.zeros((16,), jnp.float32)
        ones16 = jnp.ones((16,), jnp.float32)

        def zrow(i, carry):
            for j in range(H // 16):
                ones_v[i, pl.ds(j * 16, 16)] = zeros16
            return carry
        lax.fori_loop(0, K, zrow, 0)
        for zoff, zlen in zchunks:
            r0 = s * rows_t + zoff
            pltpu.sync_copy(ones_v.at[pl.ds(0, zlen)], deg.at[pl.ds(r0, zlen)])

        def orow(i, carry):
            for j in range(H // 16):
                ones_v[i, pl.ds(j * 16, 16)] = ones16
            return carry
        lax.fori_loop(0, K, orow, 0)
        plsc.subcore_barrier()

        base = (c * NS + s) * per_w

        def chunk(i, carry):
            off = base + i * K
            pltpu.sync_copy(dstr.at[pl.ds(off, K)], idx_d)
            pltpu.sync_copy(ones_v, deg.at[idx_d], add=True)
            return carry
        lax.fori_loop(0, n_ch, chunk, 0)
        plsc.subcore_barrier()

        r0 = s * rows_t
        pltpu.sync_copy(deg.at[pl.ds(r0, rows_t)],
                        deg_out.at[c, pl.ds(r0, rows_t)])

    return pl.kernel(
        body,
        mesh=plsc.VectorSubcoreMesh(core_axis_name="c", subcore_axis_name="s"),
        out_type=jax.ShapeDtypeStruct((NC, npad, H), jnp.float32),
        scratch_types=[
            pltpu.VMEM((K,), jnp.int32),
            pltpu.VMEM((K, H), jnp.float32),
            pltpu.VMEM_SHARED((nr, H), jnp.float32),
        ],
    )


def _rm_body(rel_ref, wm_ref, o_ref):
    o_ref[...] = jnp.dot(rel_ref[...], wm_ref[...],
                         preferred_element_type=jnp.float32)


def _dense0_body(h_ref, wm_ref, ws_ref, hm_o, hs_o):
    h = h_ref[...]
    hm_o[...] = jnp.dot(h, wm_ref[...], preferred_element_type=jnp.float32)
    hs_o[...] = jnp.dot(h, ws_ref[...], preferred_element_type=jnp.float32)


def _layer_body(agg_ref, deg_ref, hs_ref, wm_ref, ws_ref, hm_o, hs_o):
    a = agg_ref[0] + agg_ref[1]
    dg = deg_ref[0][:, :1] + deg_ref[1][:, :1]
    h = jnp.maximum(a / jnp.maximum(dg, 1.0) + hs_ref[...], 0.0)
    hm_o[...] = jnp.dot(h, wm_ref[...], preferred_element_type=jnp.float32)
    hs_o[...] = jnp.dot(h, ws_ref[...], preferred_element_type=jnp.float32)


def _final_body(agg_ref, deg_ref, hs_ref, gid_ref, t_ref, m_ref,
                out_ref, acc_ref):
    i = pl.program_id(0)

    @pl.when(i == 0)
    def _init():
        acc_ref[...] = jnp.full((G, H), -jnp.inf, jnp.float32)

    a = agg_ref[0] + agg_ref[1]
    dg = deg_ref[0][:, :1] + deg_ref[1][:, :1]
    h = jnp.maximum(a / jnp.maximum(dg, 1.0) + hs_ref[...], 0.0)
    gid = gid_ref[...]
    for g in range(G):
        hm = jnp.where(gid == g, h, -jnp.inf)
        bm = jnp.max(hm, axis=0, keepdims=True)
        acc_ref[g:g + 1, :] = jnp.maximum(acc_ref[g:g + 1, :], bm)

    @pl.when(i == pl.num_programs(0) - 1)
    def _emit():
        t = t_ref[...]
        o = jnp.full(out_ref.shape, -jnp.inf, jnp.float32)
        for g in range(G):
            o = jnp.where(t == g, acc_ref[g:g + 1, :], o)
        out_ref[...] = o * m_ref[...]


def kernel(ent_embeds, rel_embeds, W_msg, W_self, seq_mask,
           node_ids, edge_index, edge_rel, graph_ids, time_idx):
    n = node_ids.shape[0]
    e = edge_index.shape[1]
    n_rel = rel_embeds.shape[0]
    bsz, seq_len = time_idx.shape
    bt = bsz * seq_len
    npad = _cdiv(n, ROW_BLK) * ROW_BLK
    assert npad % (NW * K) == 0 or npad % NW == 0
    epad = _cdiv(e, NW * 128) * (NW * 128)
    n_blk = npad // ROW_BLK

    pad_n = npad - n
    ids_p = jnp.concatenate(
        [node_ids.astype(jnp.int32), jnp.zeros((pad_n,), jnp.int32)])
    gid_p = jnp.concatenate(
        [graph_ids.astype(jnp.int32), jnp.full((pad_n,), G, jnp.int32)]
    ).reshape(npad, 1)
    pad_e = epad + K - e
    src_p = jnp.concatenate(
        [edge_index[0].astype(jnp.int32), jnp.zeros((pad_e,), jnp.int32)])
    dst_p = jnp.concatenate(
        [edge_index[1].astype(jnp.int32), jnp.full((pad_e,), n, jnp.int32)])
    rel_p = jnp.concatenate(
        [edge_rel.astype(jnp.int32), jnp.zeros((pad_e,), jnp.int32)])
    t_p = time_idx.astype(jnp.int32).reshape(bt, 1)
    m_p = seq_mask.astype(jnp.float32).reshape(bt, 1)

    h0 = _make_gather(n, npad)(ent_embeds, ids_p)

    rm = pl.pallas_call(
        _rm_body,
        out_shape=jax.ShapeDtypeStruct((n_rel, H), jnp.float32),
    )(rel_embeds, W_msg)

    wspec = pl.BlockSpec((H, H), lambda i: (0, 0))
    rowspec = pl.BlockSpec((ROW_BLK, H), lambda i: (i, 0))
    hm0, hs0 = pl.pallas_call(
        _dense0_body,
        grid=(n_blk,),
        in_specs=[rowspec, wspec, wspec],
        out_specs=[rowspec, rowspec],
        out_shape=[jax.ShapeDtypeStruct((npad, H), jnp.float32),
                   jax.ShapeDtypeStruct((npad, H), jnp.float32)],
    )(h0, W_msg, W_self)

    scatter = _make_scatter(n, npad, epad)
    deg = _make_degree(n, npad, epad)(dst_p)
    aggspec = pl.BlockSpec((NC, ROW_BLK, H), lambda i: (0, i, 0))
    degspec = pl.BlockSpec((NC, ROW_BLK, H), lambda i: (0, i, 0))

    agg1 = scatter(hm0, rm, src_p, dst_p, rel_p)
    hm1, hs1 = pl.pallas_call(
        _layer_body,
        grid=(n_blk,),
        in_specs=[aggspec, degspec, rowspec, wspec, wspec],
        out_specs=[rowspec, rowspec],
        out_shape=[jax.ShapeDtypeStruct((npad, H), jnp.float32),
                   jax.ShapeDtypeStruct((npad, H), jnp.float32)],
    )(agg1, deg, hs0, W_msg, W_self)

    agg2 = scatter(hm1, rm, src_p, dst_p, rel_p)
    out = pl.pallas_call(
        _final_body,
        grid=(n_blk,),
        in_specs=[aggspec, degspec, rowspec,
                  pl.BlockSpec((ROW_BLK, 1), lambda i: (i, 0)),
                  pl.BlockSpec((bt, 1), lambda i: (0, 0)),
                  pl.BlockSpec((bt, 1), lambda i: (0, 0))],
        out_specs=pl.BlockSpec((bt, H), lambda i: (0, 0)),
        out_shape=jax.ShapeDtypeStruct((bt, H), jnp.float32),
        scratch_shapes=[pltpu.VMEM((G, H), jnp.float32)],
    )(agg2, deg, hs1, gid_p, t_p, m_p)

    return out.reshape(bsz, seq_len, H)


# speedup vs baseline: 1.1405x; 1.1405x over previous
import functools

import jax
import jax.numpy as jnp
from jax import lax
from jax.experimental import pallas as pl
from jax.experimental.pallas import tpu as pltpu
from jax.experimental.pallas import tpu_sc as plsc

H = 128
G = 16
ROW_BLK = 512
K = 128
NC = 2
NS = 16
NW = NC * NS


def _cdiv(a, b):
    return (a + b - 1) // b


def _make_gather(n_tbl, npad):
    rows_w = npad // NW
    ck = next(d for d in range(min(K, rows_w), 0, -8)
              if rows_w % d == 0)
    n_ch = rows_w // ck

    def body(tbl, ids, out, idx_v, rows_v, sem):
        c = lax.axis_index("c")
        s = lax.axis_index("s")
        base = (c * NS + s) * rows_w
        for i in range(n_ch):
            off = base + i * ck
            pltpu.sync_copy(ids.at[pl.ds(off, ck)], idx_v)
            pltpu.async_copy(tbl.at[idx_v], rows_v, sem).wait()
            pltpu.sync_copy(rows_v, out.at[pl.ds(off, ck)])

    return pl.kernel(
        body,
        mesh=plsc.VectorSubcoreMesh(core_axis_name="c", subcore_axis_name="s"),
        out_type=jax.ShapeDtypeStruct((npad, H), jnp.float32),
        scratch_types=[
            pltpu.VMEM((ck,), jnp.int32),
            pltpu.VMEM((ck, H), jnp.float32),
            pltpu.SemaphoreType.DMA,
        ],
    )


def _make_scatter(n, npad, epad):
    per_w = epad // NW
    n_ch = per_w // K
    nr = _cdiv(n + 1, NS * 8) * NS * 8
    rows_t = nr // NS
    zchunks = [(i * K, K) for i in range(rows_t // K)]
    if rows_t % K:
        zchunks.append((rows_t - rows_t % K, rows_t % K))

    def body(hm_tbl, rm_tbl, eidx, agg_out,
             idx3, buf_a, buf_b, sem_a, sem_b, sem_sa, sem_sb, agg):
        c = lax.axis_index("c")
        s = lax.axis_index("s")
        zeros16 = jnp.zeros((16,), jnp.float32)

        def zrow(i, carry):
            for j in range(H // 16):
                buf_a[i, pl.ds(j * 16, 16)] = zeros16
            return carry
        lax.fori_loop(0, K, zrow, 0)

        for zoff, zlen in zchunks:
            r0 = s * rows_t + zoff
            pltpu.sync_copy(buf_a.at[pl.ds(0, zlen)], agg.at[pl.ds(r0, zlen)])
        plsc.subcore_barrier()

        cbase = (c * NS + s) * n_ch

        def chunk(i, carry):
            pltpu.sync_copy(eidx.at[cbase + i], idx3)
            cp_a = pltpu.async_copy(hm_tbl.at[idx3.at[0]], buf_a, sem_a)
            cp_b = pltpu.async_copy(rm_tbl.at[idx3.at[1]], buf_b, sem_b)
            cp_a.wait()
            sc_a = pltpu.async_copy(buf_a, agg.at[idx3.at[2]], sem_sa,
                                    add=True)
            cp_b.wait()
            sc_b = pltpu.async_copy(buf_b, agg.at[idx3.at[2]], sem_sb,
                                    add=True)
            sc_a.wait()
            sc_b.wait()
            return carry
        lax.fori_loop(0, n_ch, chunk, 0)
        plsc.subcore_barrier()

        r0 = s * rows_t
        pltpu.sync_copy(agg.at[pl.ds(r0, rows_t)],
                        agg_out.at[c, pl.ds(r0, rows_t)])

    return pl.kernel(
        body,
        mesh=plsc.VectorSubcoreMesh(core_axis_name="c", subcore_axis_name="s"),
        out_type=jax.ShapeDtypeStruct((NC, npad, H), jnp.float32),
        scratch_types=[
            pltpu.VMEM((3, K), jnp.int32),
            pltpu.VMEM((K, H), jnp.float32),
            pltpu.VMEM((K, H), jnp.float32),
            pltpu.SemaphoreType.DMA,
            pltpu.SemaphoreType.DMA,
            pltpu.SemaphoreType.DMA,
            pltpu.SemaphoreType.DMA,
            pltpu.VMEM_SHARED((nr, H), jnp.float32),
        ],
    )


def _make_degree(n, npad, epad):
    per_w = epad // NW
    n_ch = per_w // K
    nr = _cdiv(n + 1, NS * 8) * NS * 8
    rows_t = nr // NS
    zchunks = [(i * K, K) for i in range(rows_t // K)]
    if rows_t % K:
        zchunks.append((rows_t - rows_t % K, rows_t % K))

    def body(dstr, deg_out, idx_d, ones_v, deg):
        c = lax.axis_index("c")
        s = lax.axis_index("s")
        zeros16 = jnp.zeros((16,), jnp.float32)
        ones16 = jnp.ones((16,), jnp.float32)

        def zrow(i, carry):
            for j in range(H // 16):
                ones_v[i, pl.ds(j * 16, 16)] = zeros16
            return carry
        lax.fori_loop(0, K, zrow, 0)
        for zoff, zlen in zchunks:
            r0 = s * rows_t + zoff
            pltpu.sync_copy(ones_v.at[pl.ds(0, zlen)], deg.at[pl.ds(r0, zlen)])

        def orow(i, carry):
            for j in range(H // 16):
                ones_v[i, pl.ds(j * 16, 16)] = ones16
            return carry
        lax.fori_loop(0, K, orow, 0)
        plsc.subcore_barrier()

        base = (c * NS + s) * per_w

        def chunk(i, carry):
            off = base + i * K
            pltpu.sync_copy(dstr.at[pl.ds(off, K)], idx_d)
            pltpu.sync_copy(ones_v, deg.at[idx_d], add=True)
            return carry
        lax.fori_loop(0, n_ch, chunk, 0)
        plsc.subcore_barrier()

        r0 = s * rows_t
        pltpu.sync_copy(deg.at[pl.ds(r0, rows_t)],
                        deg_out.at[c, pl.ds(r0, rows_t)])

    return pl.kernel(
        body,
        mesh=plsc.VectorSubcoreMesh(core_axis_name="c", subcore_axis_name="s"),
        out_type=jax.ShapeDtypeStruct((NC, npad, H), jnp.float32),
        scratch_types=[
            pltpu.VMEM((K,), jnp.int32),
            pltpu.VMEM((K, H), jnp.float32),
            pltpu.VMEM_SHARED((nr, H), jnp.float32),
        ],
    )


def _rm_body(rel_ref, wm_ref, o_ref):
    o_ref[...] = jnp.dot(rel_ref[...], wm_ref[...],
                         preferred_element_type=jnp.float32)


def _dense0_body(h_ref, wm_ref, ws_ref, hm_o, hs_o):
    h = h_ref[...]
    hm_o[...] = jnp.dot(h, wm_ref[...], preferred_element_type=jnp.float32)
    hs_o[...] = jnp.dot(h, ws_ref[...], preferred_element_type=jnp.float32)


def _layer_body(agg_ref, deg_ref, hs_ref, wm_ref, ws_ref, hm_o, hs_o):
    a = agg_ref[0] + agg_ref[1]
    dg = deg_ref[0][:, :1] + deg_ref[1][:, :1]
    h = jnp.maximum(a / jnp.maximum(dg, 1.0) + hs_ref[...], 0.0)
    hm_o[...] = jnp.dot(h, wm_ref[...], preferred_element_type=jnp.float32)
    hs_o[...] = jnp.dot(h, ws_ref[...], preferred_element_type=jnp.float32)


def _final_body(agg_ref, deg_ref, hs_ref, gid_ref, t_ref, m_ref,
                out_ref, acc_ref):
    i = pl.program_id(0)

    @pl.when(i == 0)
    def _init():
        acc_ref[...] = jnp.full((G, H), -jnp.inf, jnp.float32)

    a = agg_ref[0] + agg_ref[1]
    dg = deg_ref[0][:, :1] + deg_ref[1][:, :1]
    h = jnp.maximum(a / jnp.maximum(dg, 1.0) + hs_ref[...], 0.0)
    gid = gid_ref[...]
    for g in range(G):
        hm = jnp.where(gid == g, h, -jnp.inf)
        bm = jnp.max(hm, axis=0, keepdims=True)
        acc_ref[g:g + 1, :] = jnp.maximum(acc_ref[g:g + 1, :], bm)

    @pl.when(i == pl.num_programs(0) - 1)
    def _emit():
        t = t_ref[...]
        o = jnp.full(out_ref.shape, -jnp.inf, jnp.float32)
        for g in range(G):
            o = jnp.where(t == g, acc_ref[g:g + 1, :], o)
        out_ref[...] = o * m_ref[...]


def kernel(ent_embeds, rel_embeds, W_msg, W_self, seq_mask,
           node_ids, edge_index, edge_rel, graph_ids, time_idx):
    n = node_ids.shape[0]
    e = edge_index.shape[1]
    n_rel = rel_embeds.shape[0]
    bsz, seq_len = time_idx.shape
    bt = bsz * seq_len
    npad = _cdiv(n, ROW_BLK) * ROW_BLK
    assert npad % (NW * K) == 0 or npad % NW == 0
    epad = _cdiv(e, NW * 128) * (NW * 128)
    n_blk = npad // ROW_BLK

    pad_n = npad - n
    ids_p = jnp.concatenate(
        [node_ids.astype(jnp.int32), jnp.zeros((pad_n,), jnp.int32)])
    gid_p = jnp.concatenate(
        [graph_ids.astype(jnp.int32), jnp.full((pad_n,), G, jnp.int32)]
    ).reshape(npad, 1)
    pad_e = epad - e
    src_p = jnp.concatenate(
        [edge_index[0].astype(jnp.int32), jnp.zeros((pad_e,), jnp.int32)])
    dst_p = jnp.concatenate(
        [edge_index[1].astype(jnp.int32), jnp.full((pad_e,), n, jnp.int32)])
    rel_p = jnp.concatenate(
        [edge_rel.astype(jnp.int32), jnp.zeros((pad_e,), jnp.int32)])
    eidx = jnp.stack([src_p.reshape(-1, K), rel_p.reshape(-1, K),
                      dst_p.reshape(-1, K)], axis=1)
    t_p = time_idx.astype(jnp.int32).reshape(bt, 1)
    m_p = seq_mask.astype(jnp.float32).reshape(bt, 1)

    h0 = _make_gather(n, npad)(ent_embeds, ids_p)

    rm = pl.pallas_call(
        _rm_body,
        out_shape=jax.ShapeDtypeStruct((n_rel, H), jnp.float32),
    )(rel_embeds, W_msg)

    wspec = pl.BlockSpec((H, H), lambda i: (0, 0))
    rowspec = pl.BlockSpec((ROW_BLK, H), lambda i: (i, 0))
    hm0, hs0 = pl.pallas_call(
        _dense0_body,
        grid=(n_blk,),
        in_specs=[rowspec, wspec, wspec],
        out_specs=[rowspec, rowspec],
        out_shape=[jax.ShapeDtypeStruct((npad, H), jnp.float32),
                   jax.ShapeDtypeStruct((npad, H), jnp.float32)],
    )(h0, W_msg, W_self)

    scatter = _make_scatter(n, npad, epad)
    deg = _make_degree(n, npad, epad)(dst_p)
    aggspec = pl.BlockSpec((NC, ROW_BLK, H), lambda i: (0, i, 0))
    degspec = pl.BlockSpec((NC, ROW_BLK, H), lambda i: (0, i, 0))

    agg1 = scatter(hm0, rm, eidx)
    hm1, hs1 = pl.pallas_call(
        _layer_body,
        grid=(n_blk,),
        in_specs=[aggspec, degspec, rowspec, wspec, wspec],
        out_specs=[rowspec, rowspec],
        out_shape=[jax.ShapeDtypeStruct((npad, H), jnp.float32),
                   jax.ShapeDtypeStruct((npad, H), jnp.float32)],
    )(agg1, deg, hs0, W_msg, W_self)

    agg2 = scatter(hm1, rm, eidx)
    out = pl.pallas_call(
        _final_body,
        grid=(n_blk,),
        in_specs=[aggspec, degspec, rowspec,
                  pl.BlockSpec((ROW_BLK, 1), lambda i: (i, 0)),
                  pl.BlockSpec((bt, 1), lambda i: (0, 0)),
                  pl.BlockSpec((bt, 1), lambda i: (0, 0))],
        out_specs=pl.BlockSpec((bt, H), lambda i: (0, 0)),
        out_shape=jax.ShapeDtypeStruct((bt, H), jnp.float32),
        scratch_shapes=[pltpu.VMEM((G, H), jnp.float32)],
    )(agg2, deg, hs1, gid_p, t_p, m_p)

    return out.reshape(bsz, seq_len, H)
